# speedup vs baseline: 10.3769x; 10.3769x over previous
import jax
import jax.numpy as jnp
from jax import lax
from jax.experimental import pallas as pl
from jax.experimental.pallas import tpu as pltpu
from jax.experimental.pallas import tpu_sc as plsc

N = 10000
D = 128
H = 128
G = 64
E = 320000

NC = 2
NS = 16
NW = NC * NS
K = 128
CPW = 79
E_PAD = NW * K * CPW
N_ACC = 10240
RPT = N_ACC // NS
WD = 16

RB = 2000
NBLK = N // RB


def _sc_deg_body(dst_hbm, out_hbm, idx_v, ones_v, zb, acc):
    c = lax.axis_index("c")
    s = lax.axis_index("s")
    wid = c * NS + s

    @pl.loop(0, K)
    def _fill(i):
        zb[i, :] = jnp.zeros((WD,), jnp.float32)
        ones_v[i, :] = jnp.ones((WD,), jnp.float32)

    @pl.loop(0, RPT // K)
    def _zero(j):
        pltpu.sync_copy(zb, acc.at[pl.ds(s * RPT + j * K, K)])

    plsc.subcore_barrier()

    @pl.loop(0, CPW)
    def _edges(i):
        base = wid * (CPW * K) + i * K
        pltpu.sync_copy(dst_hbm.at[pl.ds(base, K)], idx_v)
        pltpu.sync_copy(ones_v, acc.at[idx_v], add=True)

    plsc.subcore_barrier()
    pltpu.sync_copy(acc.at[pl.ds(s * RPT, RPT)],
                    out_hbm.at[c, pl.ds(s * RPT, RPT)])


_sc_deg = pl.kernel(
    _sc_deg_body,
    out_type=jax.ShapeDtypeStruct((NC, N_ACC, WD), jnp.float32),
    mesh=plsc.VectorSubcoreMesh(core_axis_name="c", subcore_axis_name="s"),
    scratch_types=[
        pltpu.VMEM((K,), jnp.int32),
        pltpu.VMEM((K, WD), jnp.float32),
        pltpu.VMEM((K, WD), jnp.float32),
        pltpu.VMEM_SHARED((N_ACC, WD), jnp.float32),
    ],
)


def _sc_agg_body(g_hbm, src_hbm, dst_hbm, out_hbm, idx_s, idx_d, rows, zb,
                 acc, sem):
    c = lax.axis_index("c")
    s = lax.axis_index("s")
    wid = c * NS + s

    @pl.loop(0, K)
    def _fill(i):
        for j in range(H // 16):
            zb[i, pl.ds(j * 16, 16)] = jnp.zeros((16,), jnp.float32)

    @pl.loop(0, RPT // K)
    def _zero(j):
        pltpu.sync_copy(zb, acc.at[pl.ds(s * RPT + j * K, K)])

    plsc.subcore_barrier()

    @pl.loop(0, CPW)
    def _edges(i):
        base = wid * (CPW * K) + i * K
        pltpu.sync_copy(src_hbm.at[pl.ds(base, K)], idx_s)
        pltpu.sync_copy(dst_hbm.at[pl.ds(base, K)], idx_d)
        pltpu.async_copy(g_hbm.at[idx_s], rows, sem).wait()
        pltpu.sync_copy(rows, acc.at[idx_d], add=True)

    plsc.subcore_barrier()
    pltpu.sync_copy(acc.at[pl.ds(s * RPT, RPT)],
                    out_hbm.at[c, pl.ds(s * RPT, RPT)])


_sc_agg = pl.kernel(
    _sc_agg_body,
    out_type=jax.ShapeDtypeStruct((NC, N_ACC, H), jnp.float32),
    mesh=plsc.VectorSubcoreMesh(core_axis_name="c", subcore_axis_name="s"),
    scratch_types=[
        pltpu.VMEM((K,), jnp.int32),
        pltpu.VMEM((K,), jnp.int32),
        pltpu.VMEM((K, H), jnp.float32),
        pltpu.VMEM((K, H), jnp.float32),
        pltpu.VMEM_SHARED((N_ACC, H), jnp.float32),
        pltpu.SemaphoreType.DMA,
    ],
)


def _tc_scale_body(x_ref, w1_ref, degp_ref, g1_ref, dinv_ref):
    deg = degp_ref[0, :, 0:1] + degp_ref[1, :, 0:1] + 1.0
    dinv = lax.rsqrt(deg)
    h1 = jnp.dot(x_ref[...], w1_ref[...], preferred_element_type=jnp.float32)
    g1_ref[...] = h1 * dinv
    dinv_ref[...] = dinv


_tc_scale = pl.pallas_call(
    _tc_scale_body,
    grid=(NBLK,),
    in_specs=[
        pl.BlockSpec((RB, D), lambda i: (i, 0)),
        pl.BlockSpec((D, H), lambda i: (0, 0)),
        pl.BlockSpec((NC, RB, WD), lambda i: (0, i, 0)),
    ],
    out_specs=[
        pl.BlockSpec((RB, H), lambda i: (i, 0)),
        pl.BlockSpec((RB, 1), lambda i: (i, 0)),
    ],
    out_shape=[
        jax.ShapeDtypeStruct((N, H), jnp.float32),
        jax.ShapeDtypeStruct((N, 1), jnp.float32),
    ],
)


def _tc_mid_body(aggp_ref, g1_ref, dinv_ref, b1_ref, w2_ref, g2_ref):
    agg = aggp_ref[0] + aggp_ref[1] + g1_ref[...]
    dinv = dinv_ref[...]
    out1 = jnp.maximum(agg * dinv + b1_ref[...], 0.0)
    h2 = jnp.dot(out1, w2_ref[...], preferred_element_type=jnp.float32)
    g2_ref[...] = h2 * dinv


_tc_mid = pl.pallas_call(
    _tc_mid_body,
    grid=(NBLK,),
    in_specs=[
        pl.BlockSpec((NC, RB, H), lambda i: (0, i, 0)),
        pl.BlockSpec((RB, H), lambda i: (i, 0)),
        pl.BlockSpec((RB, 1), lambda i: (i, 0)),
        pl.BlockSpec((1, H), lambda i: (0, 0)),
        pl.BlockSpec((H, H), lambda i: (0, 0)),
    ],
    out_specs=pl.BlockSpec((RB, H), lambda i: (i, 0)),
    out_shape=jax.ShapeDtypeStruct((N, H), jnp.float32),
)


def _tc_pool_body(aggp_ref, g2_ref, dinv_ref, b2_ref, batch_ref, wfc_ref,
                  bfc_ref, out_ref, s_acc, c_acc):
    i = pl.program_id(0)
    agg = aggp_ref[0] + aggp_ref[1] + g2_ref[...]
    out2 = jnp.maximum(agg * dinv_ref[...] + b2_ref[...], 0.0)
    seg = (batch_ref[...] == lax.broadcasted_iota(jnp.int32, (RB, G), 1))
    seg = seg.astype(jnp.float32)
    part_s = lax.dot_general(seg, out2, (((0,), (0,)), ((), ())),
                             preferred_element_type=jnp.float32)
    ones = jnp.ones((RB, 1), jnp.float32)
    part_c = lax.dot_general(seg, ones, (((0,), (0,)), ((), ())),
                             preferred_element_type=jnp.float32)

    @pl.when(i == 0)
    def _():
        s_acc[...] = part_s
        c_acc[...] = part_c

    @pl.when(i > 0)
    def _():
        s_acc[...] += part_s
        c_acc[...] += part_c

    @pl.when(i == NBLK - 1)
    def _():
        pooled = s_acc[...] / jnp.maximum(c_acc[...], 1.0)
        out_ref[...] = (jnp.dot(pooled, wfc_ref[...],
                                preferred_element_type=jnp.float32)
                        + bfc_ref[...])


_tc_pool = pl.pallas_call(
    _tc_pool_body,
    grid=(NBLK,),
    in_specs=[
        pl.BlockSpec((NC, RB, H), lambda i: (0, i, 0)),
        pl.BlockSpec((RB, H), lambda i: (i, 0)),
        pl.BlockSpec((RB, 1), lambda i: (i, 0)),
        pl.BlockSpec((1, H), lambda i: (0, 0)),
        pl.BlockSpec((RB, 1), lambda i: (i, 0)),
        pl.BlockSpec((H, 1), lambda i: (0, 0)),
        pl.BlockSpec((1, 1), lambda i: (0, 0)),
    ],
    out_specs=pl.BlockSpec((G, 1), lambda i: (0, 0)),
    out_shape=jax.ShapeDtypeStruct((G, 1), jnp.float32),
    scratch_shapes=[
        pltpu.VMEM((G, H), jnp.float32),
        pltpu.VMEM((G, 1), jnp.float32),
    ],
)


def kernel(x, edge_index, batch, W1, b1, W2, b2, Wfc, bfc):
    pad = E_PAD - E
    srcp = jnp.concatenate([edge_index[0], jnp.zeros((pad,), jnp.int32)])
    dstp = jnp.concatenate([edge_index[1], jnp.full((pad,), N, jnp.int32)])

    degp = _sc_deg(dstp)
    g1, dinv = _tc_scale(x, W1, degp)
    agg1 = _sc_agg(g1, srcp, dstp)
    g2 = _tc_mid(agg1, g1, dinv, b1.reshape(1, H), W2)
    agg2 = _sc_agg(g2, srcp, dstp)
    out = _tc_pool(agg2, g2, dinv, b2.reshape(1, H), batch.reshape(N, 1),
                   Wfc, bfc.reshape(1, 1))
    return out.reshape(G)
